# Initial kernel scaffold; baseline (speedup 1.0000x reference)
import functools

import jax
import jax.numpy as jnp
from jax import lax
from jax.experimental import pallas as pl
from jax.experimental.pallas import tpu as pltpu
from jax.experimental.pallas import tpu_sc as plsc

H = 32
_NW = 32
_W = 128


def _silu(x):
    return x * jax.nn.sigmoid(x)


def _sc_gather_pair(table, src, dest):
    ep = src.shape[0]
    d = table.shape[1]
    per_w = ep // _NW
    nwin = per_w // _W
    mesh = plsc.VectorSubcoreMesh(core_axis_name="c", subcore_axis_name="s")

    @functools.partial(
        pl.kernel,
        mesh=mesh,
        out_type=(jax.ShapeDtypeStruct((ep, d), jnp.float32),
                  jax.ShapeDtypeStruct((ep, d), jnp.float32)),
        scratch_types=[
            pltpu.VMEM((_W,), jnp.int32),
            pltpu.VMEM((_W,), jnp.int32),
            pltpu.VMEM((_W, d), jnp.float32),
            pltpu.VMEM((_W, d), jnp.float32),
            pltpu.SemaphoreType.DMA,
            pltpu.SemaphoreType.DMA,
        ],
    )
    def k(table_hbm, src_hbm, dest_hbm, outs_hbm, outd_hbm,
          sidx_v, didx_v, srows_v, drows_v, sem_s, sem_d):
        wid = lax.axis_index("s") * 2 + lax.axis_index("c")
        base = wid * per_w

        def body(w, carry):
            off = base + w * _W
            pltpu.sync_copy(src_hbm.at[pl.ds(off, _W)], sidx_v)
            pltpu.sync_copy(dest_hbm.at[pl.ds(off, _W)], didx_v)
            cp_s = pltpu.async_copy(table_hbm.at[sidx_v], srows_v, sem_s)
            cp_d = pltpu.async_copy(table_hbm.at[didx_v], drows_v, sem_d)
            cp_s.wait()
            cp_d.wait()
            pltpu.sync_copy(srows_v, outs_hbm.at[pl.ds(off, _W)])
            pltpu.sync_copy(drows_v, outd_hbm.at[pl.ds(off, _W)])
            return carry

        lax.fori_loop(0, nwin, body, 0)

    return k(table, src, dest)


def _sc_scatter_add(upd, didx, zeros, n_nodes):
    ep = upd.shape[0]
    nh = n_nodes // 2
    nhp = nh + 16
    per_t = ep // 16
    nwin = per_t // _W
    zrows = nhp // 16
    orows = nh // 16
    mesh = plsc.VectorSubcoreMesh(core_axis_name="c", subcore_axis_name="s")

    @functools.partial(
        pl.kernel,
        mesh=mesh,
        out_type=jax.ShapeDtypeStruct((n_nodes, H), jnp.float32),
        scratch_types=[
            pltpu.VMEM((_W,), jnp.int32),
            pltpu.VMEM((_W,), jnp.int32),
            pltpu.VMEM((_W, H), jnp.float32),
            pltpu.VMEM_SHARED((nhp, H), jnp.float32),
            pltpu.SemaphoreType.DMA,
        ],
    )
    def k(upd_hbm, idx_hbm, z_hbm, out_hbm,
          idx_v, lidx_v, rows_v, acc_sh, sem):
        cid = lax.axis_index("c")
        sid = lax.axis_index("s")
        nbase = cid * nh
        pltpu.sync_copy(z_hbm, acc_sh.at[pl.ds(sid * zrows, zrows)])
        plsc.subcore_barrier()

        ebase = sid * per_t

        def body(w, carry):
            off = ebase + w * _W
            pltpu.sync_copy(idx_hbm.at[pl.ds(off, _W)], idx_v)
            cp = pltpu.async_copy(upd_hbm.at[pl.ds(off, _W)], rows_v, sem)

            def ibody(i, c2):
                v = idx_v[pl.ds(i * 16, 16)]
                ok = (v >= nbase) & (v < nbase + nh)
                lidx_v[pl.ds(i * 16, 16)] = jnp.where(ok, v - nbase, nh + sid)
                return c2

            lax.fori_loop(0, _W // 16, ibody, 0)
            cp.wait()
            pltpu.sync_copy(rows_v, acc_sh.at[lidx_v], add=True)
            return carry

        lax.fori_loop(0, nwin, body, 0)
        plsc.subcore_barrier()
        pltpu.sync_copy(acc_sh.at[pl.ds(sid * orows, orows)],
                        out_hbm.at[pl.ds(nbase + sid * orows, orows)])

    return k(upd, didx, zeros)


def _full(shape):
    return pl.BlockSpec(shape, lambda i: (0, 0))


def _rows(blk, width):
    return pl.BlockSpec((blk, width), lambda i: (i, 0))


def _node_encode(zn, ps, blk):
    (w1, b1), (w2, b2), (w3, b3) = ps
    n = zn.shape[0]

    def body(zn_r, w1_r, b1_r, w2_r, b2_r, w3_r, b3_r, o_r):
        h = _silu(jnp.dot(zn_r[...], w1_r[...],
                          preferred_element_type=jnp.float32) + b1_r[...])
        h = _silu(jnp.dot(h, w2_r[...],
                          preferred_element_type=jnp.float32) + b2_r[...])
        o_r[...] = jnp.dot(h, w3_r[...],
                           preferred_element_type=jnp.float32) + b3_r[...]

    return pl.pallas_call(
        body,
        grid=(n // blk,),
        in_specs=[_rows(blk, 6),
                  _full((6, H)), _full((1, H)),
                  _full((H, H)), _full((1, H)),
                  _full((H, H)), _full((1, H))],
        out_specs=_rows(blk, H),
        out_shape=jax.ShapeDtypeStruct((n, H), jnp.float32),
    )(zn, w1, b1.reshape(1, H), w2, b2.reshape(1, H), w3, b3.reshape(1, H))


def _edge_pass0(gqs, gqd, gxs, gxd, enc_ps, e_ps, blk):
    (ew1, eb1), (ew2, eb2), (ew3, eb3) = enc_ps
    (pw1, pb1), (pw2, pb2), (pw3, pb3) = e_ps
    ep = gqs.shape[0]
    ew1q = jnp.zeros((16, H), jnp.float32).at[:3].set(ew1[:3])
    ew1n = ew1[3:4]
    pwa, pwb, pwc = pw1[0:H], pw1[H:2 * H], pw1[2 * H:3 * H]

    def body(gqs_r, gqd_r, gxs_r, gxd_r,
             ew1q_r, ew1n_r, eb1_r, ew2_r, eb2_r, ew3_r, eb3_r,
             pwa_r, pwb_r, pwc_r, pb1_r, pw2_r, pb2_r, pw3_r, pb3_r,
             ean_r, ea1_r):
        d = gqs_r[...] - gqd_r[...]
        un = jnp.sqrt(jnp.sum(d * d, axis=1, keepdims=True))
        h = jnp.dot(d, ew1q_r[...], preferred_element_type=jnp.float32)
        h = _silu(h + un * ew1n_r[...] + eb1_r[...])
        h = _silu(jnp.dot(h, ew2_r[...],
                          preferred_element_type=jnp.float32) + eb2_r[...])
        ea = jnp.dot(h, ew3_r[...],
                     preferred_element_type=jnp.float32) + eb3_r[...]
        g = (jnp.dot(ea, pwa_r[...], preferred_element_type=jnp.float32)
             + jnp.dot(gxs_r[...], pwb_r[...], preferred_element_type=jnp.float32)
             + jnp.dot(gxd_r[...], pwc_r[...], preferred_element_type=jnp.float32)
             + pb1_r[...])
        g = _silu(g)
        g = _silu(jnp.dot(g, pw2_r[...],
                          preferred_element_type=jnp.float32) + pb2_r[...])
        ean = jnp.dot(g, pw3_r[...],
                      preferred_element_type=jnp.float32) + pb3_r[...]
        ean_r[...] = ean
        ea1_r[...] = ea + ean

    return pl.pallas_call(
        body,
        grid=(ep // blk,),
        in_specs=[_rows(blk, 16), _rows(blk, 16), _rows(blk, H), _rows(blk, H),
                  _full((16, H)), _full((1, H)), _full((1, H)),
                  _full((H, H)), _full((1, H)), _full((H, H)), _full((1, H)),
                  _full((H, H)), _full((H, H)), _full((H, H)), _full((1, H)),
                  _full((H, H)), _full((1, H)), _full((H, H)), _full((1, H))],
        out_specs=(_rows(blk, H), _rows(blk, H)),
        out_shape=(jax.ShapeDtypeStruct((ep, H), jnp.float32),
                   jax.ShapeDtypeStruct((ep, H), jnp.float32)),
    )(gqs, gqd, gxs, gxd,
      ew1q, ew1n, eb1.reshape(1, H), ew2, eb2.reshape(1, H), ew3,
      eb3.reshape(1, H), pwa, pwb, pwc, pb1.reshape(1, H), pw2,
      pb2.reshape(1, H), pw3, pb3.reshape(1, H))


def _edge_pass1(ea, gxs, gxd, e_ps, blk):
    (pw1, pb1), (pw2, pb2), (pw3, pb3) = e_ps
    ep = ea.shape[0]
    pwa, pwb, pwc = pw1[0:H], pw1[H:2 * H], pw1[2 * H:3 * H]

    def body(ea_r, gxs_r, gxd_r, pwa_r, pwb_r, pwc_r, pb1_r,
             pw2_r, pb2_r, pw3_r, pb3_r, ean_r):
        g = (jnp.dot(ea_r[...], pwa_r[...], preferred_element_type=jnp.float32)
             + jnp.dot(gxs_r[...], pwb_r[...], preferred_element_type=jnp.float32)
             + jnp.dot(gxd_r[...], pwc_r[...], preferred_element_type=jnp.float32)
             + pb1_r[...])
        g = _silu(g)
        g = _silu(jnp.dot(g, pw2_r[...],
                          preferred_element_type=jnp.float32) + pb2_r[...])
        ean_r[...] = jnp.dot(g, pw3_r[...],
                             preferred_element_type=jnp.float32) + pb3_r[...]

    return pl.pallas_call(
        body,
        grid=(ep // blk,),
        in_specs=[_rows(blk, H), _rows(blk, H), _rows(blk, H),
                  _full((H, H)), _full((H, H)), _full((H, H)), _full((1, H)),
                  _full((H, H)), _full((1, H)), _full((H, H)), _full((1, H))],
        out_specs=_rows(blk, H),
        out_shape=jax.ShapeDtypeStruct((ep, H), jnp.float32),
    )(ea, gxs, gxd, pwa, pwb, pwc, pb1.reshape(1, H), pw2,
      pb2.reshape(1, H), pw3, pb3.reshape(1, H))


def _node_update(x, agg, n_ps, dec_ps, blk):
    (nw1, nb1), (nw2, nb2), (nw3, nb3) = n_ps
    nwa, nwb = nw1[0:H], nw1[H:2 * H]
    n = x.shape[0]
    if dec_ps is None:
        wdim = H

        def body(x_r, agg_r, nwa_r, nwb_r, nb1_r, nw2_r, nb2_r, nw3_r, nb3_r,
                 o_r):
            g = (jnp.dot(x_r[...], nwa_r[...], preferred_element_type=jnp.float32)
                 + jnp.dot(agg_r[...], nwb_r[...], preferred_element_type=jnp.float32)
                 + nb1_r[...])
            g = _silu(g)
            g = _silu(jnp.dot(g, nw2_r[...],
                              preferred_element_type=jnp.float32) + nb2_r[...])
            o_r[...] = x_r[...] + jnp.dot(
                g, nw3_r[...], preferred_element_type=jnp.float32) + nb3_r[...]

        extra_in, extra_args = [], []
    else:
        (dw1, db1), (dw2, db2) = dec_ps
        wdim = 3

        def body(x_r, agg_r, nwa_r, nwb_r, nb1_r, nw2_r, nb2_r, nw3_r, nb3_r,
                 dw1_r, db1_r, dw2_r, db2_r, o_r):
            g = (jnp.dot(x_r[...], nwa_r[...], preferred_element_type=jnp.float32)
                 + jnp.dot(agg_r[...], nwb_r[...], preferred_element_type=jnp.float32)
                 + nb1_r[...])
            g = _silu(g)
            g = _silu(jnp.dot(g, nw2_r[...],
                              preferred_element_type=jnp.float32) + nb2_r[...])
            xo = x_r[...] + jnp.dot(
                g, nw3_r[...], preferred_element_type=jnp.float32) + nb3_r[...]
            y = _silu(jnp.dot(xo, dw1_r[...],
                              preferred_element_type=jnp.float32) + db1_r[...])
            o_r[...] = jnp.dot(y, dw2_r[...],
                               preferred_element_type=jnp.float32) + db2_r[...]

        extra_in = [_full((H, H)), _full((1, H)), _full((H, 3)), _full((1, 3))]
        extra_args = [dw1, db1.reshape(1, H), dw2, db2.reshape(1, 3)]

    return pl.pallas_call(
        body,
        grid=(n // blk,),
        in_specs=[_rows(blk, H), _rows(blk, H),
                  _full((H, H)), _full((H, H)), _full((1, H)),
                  _full((H, H)), _full((1, H)), _full((H, H)), _full((1, H))]
                 + extra_in,
        out_specs=_rows(blk, wdim),
        out_shape=jax.ShapeDtypeStruct((n, wdim), jnp.float32),
    )(x, agg, nwa, nwb, nb1.reshape(1, H), nw2, nb2.reshape(1, H), nw3,
      nb3.reshape(1, H), *extra_args)


def kernel(z, n, edge_index, q_0, params):
    nn = z.shape[0]
    e = edge_index.shape[1]
    chunk = _NW * _W
    ep = ((e + chunk - 1) // chunk) * chunk

    src = edge_index[0]
    dest = edge_index[1]
    src_p = jnp.concatenate([src, jnp.zeros((ep - e,), jnp.int32)])
    dest_p = jnp.concatenate([dest, jnp.full((ep - e,), nn, jnp.int32)])

    zn = jnp.concatenate([z, n], axis=1)
    q0p = jnp.pad(q_0, ((0, 0), (0, 13)))
    zeros = jnp.zeros(((nn // 2 + 16) // 16, H), jnp.float32)

    blk_n = 2000
    blk_e = 4096

    x0 = _node_encode(zn, params["enc_node"], blk_n)
    gqs, gqd = _sc_gather_pair(q0p, src_p, dest_p)
    gxs, gxd = _sc_gather_pair(x0, src_p, dest_p)
    ea0, eattr1 = _edge_pass0(gqs, gqd, gxs, gxd,
                              params["enc_edge"], params["edge_0"], blk_e)
    agg0 = _sc_scatter_add(ea0, dest_p, zeros, nn)
    x1 = _node_update(x0, agg0, params["node_0"], None, blk_n)
    gxs1, gxd1 = _sc_gather_pair(x1, src_p, dest_p)
    ea1 = _edge_pass1(eattr1, gxs1, gxd1, params["edge_1"], blk_e)
    agg1 = _sc_scatter_add(ea1, dest_p, zeros, nn)
    return _node_update(x1, agg1, params["node_1"], params["dec"], blk_n)


# speedup vs baseline: 2.6062x; 2.6062x over previous
import functools

import jax
import jax.numpy as jnp
from jax import lax
from jax.experimental import pallas as pl
from jax.experimental.pallas import tpu as pltpu
from jax.experimental.pallas import tpu_sc as plsc

H = 32
_NW = 32
_W = 128


def _silu(x):
    return x * jax.nn.sigmoid(x)


def _sc_gather_pair(table, src, dest):
    ep = src.shape[0]
    d = table.shape[1]
    per_w = ep // _NW
    nwin = per_w // _W
    mesh = plsc.VectorSubcoreMesh(core_axis_name="c", subcore_axis_name="s")

    @functools.partial(
        pl.kernel,
        mesh=mesh,
        out_type=(jax.ShapeDtypeStruct((ep, d), jnp.float32),
                  jax.ShapeDtypeStruct((ep, d), jnp.float32)),
        scratch_types=[
            pltpu.VMEM((_W,), jnp.int32),
            pltpu.VMEM((_W,), jnp.int32),
            pltpu.VMEM((_W, d), jnp.float32),
            pltpu.VMEM((_W, d), jnp.float32),
            pltpu.SemaphoreType.DMA,
            pltpu.SemaphoreType.DMA,
        ],
        compiler_params=pltpu.CompilerParams(use_tc_tiling_on_sc=False),
    )
    def k(table_hbm, src_hbm, dest_hbm, outs_hbm, outd_hbm,
          sidx_v, didx_v, srows_v, drows_v, sem_s, sem_d):
        wid = lax.axis_index("s") * 2 + lax.axis_index("c")
        base = wid * per_w

        def body(w, carry):
            off = base + w * _W
            pltpu.sync_copy(src_hbm.at[pl.ds(off, _W)], sidx_v)
            pltpu.sync_copy(dest_hbm.at[pl.ds(off, _W)], didx_v)
            cp_s = pltpu.async_copy(table_hbm.at[sidx_v], srows_v, sem_s)
            cp_d = pltpu.async_copy(table_hbm.at[didx_v], drows_v, sem_d)
            cp_s.wait()
            cp_d.wait()
            pltpu.sync_copy(srows_v, outs_hbm.at[pl.ds(off, _W)])
            pltpu.sync_copy(drows_v, outd_hbm.at[pl.ds(off, _W)])
            return carry

        lax.fori_loop(0, nwin, body, 0)

    return k(table, src, dest)


def _sc_scatter_add(upd, didx, zeros, n_nodes):
    ep = upd.shape[0]
    nh = n_nodes // 2
    nhp = nh + 16
    per_t = ep // 16
    nwin = per_t // _W
    zrows = nhp // 16
    orows = nh // 16
    mesh = plsc.VectorSubcoreMesh(core_axis_name="c", subcore_axis_name="s")

    @functools.partial(
        pl.kernel,
        mesh=mesh,
        out_type=jax.ShapeDtypeStruct((n_nodes, H), jnp.float32),
        scratch_types=[
            pltpu.VMEM((_W,), jnp.int32),
            pltpu.VMEM((_W,), jnp.int32),
            pltpu.VMEM((_W, H), jnp.float32),
            pltpu.VMEM_SHARED((nhp, H), jnp.float32),
            pltpu.SemaphoreType.DMA,
        ],
        compiler_params=pltpu.CompilerParams(use_tc_tiling_on_sc=False),
    )
    def k(upd_hbm, idx_hbm, z_hbm, out_hbm,
          idx_v, lidx_v, rows_v, acc_sh, sem):
        cid = lax.axis_index("c")
        sid = lax.axis_index("s")
        nbase = cid * nh
        pltpu.sync_copy(z_hbm, acc_sh.at[pl.ds(sid * zrows, zrows)])
        plsc.subcore_barrier()

        ebase = sid * per_t

        def body(w, carry):
            off = ebase + w * _W
            pltpu.sync_copy(idx_hbm.at[pl.ds(off, _W)], idx_v)
            cp = pltpu.async_copy(upd_hbm.at[pl.ds(off, _W)], rows_v, sem)

            def ibody(i, c2):
                v = idx_v[pl.ds(i * 16, 16)]
                ok = (v >= nbase) & (v < nbase + nh)
                lidx_v[pl.ds(i * 16, 16)] = jnp.where(ok, v - nbase, nh + sid)
                return c2

            lax.fori_loop(0, _W // 16, ibody, 0)
            cp.wait()
            pltpu.sync_copy(rows_v, acc_sh.at[lidx_v], add=True)
            return carry

        lax.fori_loop(0, nwin, body, 0)
        plsc.subcore_barrier()
        pltpu.sync_copy(acc_sh.at[pl.ds(sid * orows, orows)],
                        out_hbm.at[pl.ds(nbase + sid * orows, orows)])

    return k(upd, didx, zeros)


def _full(shape):
    return pl.BlockSpec(shape, lambda i: (0, 0))


def _rows(blk, width):
    return pl.BlockSpec((blk, width), lambda i: (i, 0))


def _node_encode(zn, ps, blk):
    (w1, b1), (w2, b2), (w3, b3) = ps
    n = zn.shape[0]

    def body(zn_r, w1_r, b1_r, w2_r, b2_r, w3_r, b3_r, o_r):
        h = _silu(jnp.dot(zn_r[...], w1_r[...],
                          preferred_element_type=jnp.float32) + b1_r[...])
        h = _silu(jnp.dot(h, w2_r[...],
                          preferred_element_type=jnp.float32) + b2_r[...])
        o_r[...] = jnp.dot(h, w3_r[...],
                           preferred_element_type=jnp.float32) + b3_r[...]

    return pl.pallas_call(
        body,
        grid=(n // blk,),
        in_specs=[_rows(blk, 6),
                  _full((6, H)), _full((1, H)),
                  _full((H, H)), _full((1, H)),
                  _full((H, H)), _full((1, H))],
        out_specs=_rows(blk, H),
        out_shape=jax.ShapeDtypeStruct((n, H), jnp.float32),
    )(zn, w1, b1.reshape(1, H), w2, b2.reshape(1, H), w3, b3.reshape(1, H))


def _edge_pass0(gqs, gqd, gxs, gxd, enc_ps, e_ps, blk):
    (ew1, eb1), (ew2, eb2), (ew3, eb3) = enc_ps
    (pw1, pb1), (pw2, pb2), (pw3, pb3) = e_ps
    ep = gqs.shape[0]
    ew1q = jnp.zeros((16, H), jnp.float32).at[:3].set(ew1[:3])
    ew1n = ew1[3:4]
    pwa, pwb, pwc = pw1[0:H], pw1[H:2 * H], pw1[2 * H:3 * H]

    def body(gqs_r, gqd_r, gxs_r, gxd_r,
             ew1q_r, ew1n_r, eb1_r, ew2_r, eb2_r, ew3_r, eb3_r,
             pwa_r, pwb_r, pwc_r, pb1_r, pw2_r, pb2_r, pw3_r, pb3_r,
             ean_r, ea1_r):
        d = gqs_r[...] - gqd_r[...]
        un = jnp.sqrt(jnp.sum(d * d, axis=1, keepdims=True))
        h = jnp.dot(d, ew1q_r[...], preferred_element_type=jnp.float32)
        h = _silu(h + un * ew1n_r[...] + eb1_r[...])
        h = _silu(jnp.dot(h, ew2_r[...],
                          preferred_element_type=jnp.float32) + eb2_r[...])
        ea = jnp.dot(h, ew3_r[...],
                     preferred_element_type=jnp.float32) + eb3_r[...]
        g = (jnp.dot(ea, pwa_r[...], preferred_element_type=jnp.float32)
             + jnp.dot(gxs_r[...], pwb_r[...], preferred_element_type=jnp.float32)
             + jnp.dot(gxd_r[...], pwc_r[...], preferred_element_type=jnp.float32)
             + pb1_r[...])
        g = _silu(g)
        g = _silu(jnp.dot(g, pw2_r[...],
                          preferred_element_type=jnp.float32) + pb2_r[...])
        ean = jnp.dot(g, pw3_r[...],
                      preferred_element_type=jnp.float32) + pb3_r[...]
        ean_r[...] = ean
        ea1_r[...] = ea + ean

    return pl.pallas_call(
        body,
        grid=(ep // blk,),
        in_specs=[_rows(blk, 16), _rows(blk, 16), _rows(blk, H), _rows(blk, H),
                  _full((16, H)), _full((1, H)), _full((1, H)),
                  _full((H, H)), _full((1, H)), _full((H, H)), _full((1, H)),
                  _full((H, H)), _full((H, H)), _full((H, H)), _full((1, H)),
                  _full((H, H)), _full((1, H)), _full((H, H)), _full((1, H))],
        out_specs=(_rows(blk, H), _rows(blk, H)),
        out_shape=(jax.ShapeDtypeStruct((ep, H), jnp.float32),
                   jax.ShapeDtypeStruct((ep, H), jnp.float32)),
    )(gqs, gqd, gxs, gxd,
      ew1q, ew1n, eb1.reshape(1, H), ew2, eb2.reshape(1, H), ew3,
      eb3.reshape(1, H), pwa, pwb, pwc, pb1.reshape(1, H), pw2,
      pb2.reshape(1, H), pw3, pb3.reshape(1, H))


def _edge_pass1(ea, gxs, gxd, e_ps, blk):
    (pw1, pb1), (pw2, pb2), (pw3, pb3) = e_ps
    ep = ea.shape[0]
    pwa, pwb, pwc = pw1[0:H], pw1[H:2 * H], pw1[2 * H:3 * H]

    def body(ea_r, gxs_r, gxd_r, pwa_r, pwb_r, pwc_r, pb1_r,
             pw2_r, pb2_r, pw3_r, pb3_r, ean_r):
        g = (jnp.dot(ea_r[...], pwa_r[...], preferred_element_type=jnp.float32)
             + jnp.dot(gxs_r[...], pwb_r[...], preferred_element_type=jnp.float32)
             + jnp.dot(gxd_r[...], pwc_r[...], preferred_element_type=jnp.float32)
             + pb1_r[...])
        g = _silu(g)
        g = _silu(jnp.dot(g, pw2_r[...],
                          preferred_element_type=jnp.float32) + pb2_r[...])
        ean_r[...] = jnp.dot(g, pw3_r[...],
                             preferred_element_type=jnp.float32) + pb3_r[...]

    return pl.pallas_call(
        body,
        grid=(ep // blk,),
        in_specs=[_rows(blk, H), _rows(blk, H), _rows(blk, H),
                  _full((H, H)), _full((H, H)), _full((H, H)), _full((1, H)),
                  _full((H, H)), _full((1, H)), _full((H, H)), _full((1, H))],
        out_specs=_rows(blk, H),
        out_shape=jax.ShapeDtypeStruct((ep, H), jnp.float32),
    )(ea, gxs, gxd, pwa, pwb, pwc, pb1.reshape(1, H), pw2,
      pb2.reshape(1, H), pw3, pb3.reshape(1, H))


def _node_update(x, agg, n_ps, dec_ps, blk):
    (nw1, nb1), (nw2, nb2), (nw3, nb3) = n_ps
    nwa, nwb = nw1[0:H], nw1[H:2 * H]
    n = x.shape[0]
    if dec_ps is None:
        wdim = H

        def body(x_r, agg_r, nwa_r, nwb_r, nb1_r, nw2_r, nb2_r, nw3_r, nb3_r,
                 o_r):
            g = (jnp.dot(x_r[...], nwa_r[...], preferred_element_type=jnp.float32)
                 + jnp.dot(agg_r[...], nwb_r[...], preferred_element_type=jnp.float32)
                 + nb1_r[...])
            g = _silu(g)
            g = _silu(jnp.dot(g, nw2_r[...],
                              preferred_element_type=jnp.float32) + nb2_r[...])
            o_r[...] = x_r[...] + jnp.dot(
                g, nw3_r[...], preferred_element_type=jnp.float32) + nb3_r[...]

        extra_in, extra_args = [], []
    else:
        (dw1, db1), (dw2, db2), (dw3, db3) = dec_ps
        wdim = 3

        def body(x_r, agg_r, nwa_r, nwb_r, nb1_r, nw2_r, nb2_r, nw3_r, nb3_r,
                 dw1_r, db1_r, dw2_r, db2_r, dw3_r, db3_r, o_r):
            g = (jnp.dot(x_r[...], nwa_r[...], preferred_element_type=jnp.float32)
                 + jnp.dot(agg_r[...], nwb_r[...], preferred_element_type=jnp.float32)
                 + nb1_r[...])
            g = _silu(g)
            g = _silu(jnp.dot(g, nw2_r[...],
                              preferred_element_type=jnp.float32) + nb2_r[...])
            xo = x_r[...] + jnp.dot(
                g, nw3_r[...], preferred_element_type=jnp.float32) + nb3_r[...]
            y = _silu(jnp.dot(xo, dw1_r[...],
                              preferred_element_type=jnp.float32) + db1_r[...])
            y = _silu(jnp.dot(y, dw2_r[...],
                              preferred_element_type=jnp.float32) + db2_r[...])
            o_r[...] = jnp.dot(y, dw3_r[...],
                               preferred_element_type=jnp.float32) + db3_r[...]

        extra_in = [_full((H, H)), _full((1, H)), _full((H, H)), _full((1, H)),
                    _full((H, 3)), _full((1, 3))]
        extra_args = [dw1, db1.reshape(1, H), dw2, db2.reshape(1, H),
                      dw3, db3.reshape(1, 3)]

    return pl.pallas_call(
        body,
        grid=(n // blk,),
        in_specs=[_rows(blk, H), _rows(blk, H),
                  _full((H, H)), _full((H, H)), _full((1, H)),
                  _full((H, H)), _full((1, H)), _full((H, H)), _full((1, H))]
                 + extra_in,
        out_specs=_rows(blk, wdim),
        out_shape=jax.ShapeDtypeStruct((n, wdim), jnp.float32),
    )(x, agg, nwa, nwb, nb1.reshape(1, H), nw2, nb2.reshape(1, H), nw3,
      nb3.reshape(1, H), *extra_args)


def kernel(z, n, edge_index, q_0, params):
    nn = z.shape[0]
    e = edge_index.shape[1]
    chunk = _NW * _W
    ep = ((e + chunk - 1) // chunk) * chunk

    src = edge_index[0]
    dest = edge_index[1]
    src_p = jnp.concatenate([src, jnp.zeros((ep - e,), jnp.int32)])
    dest_p = jnp.concatenate([dest, jnp.zeros((ep - e,), jnp.int32)])
    dest_s = jnp.concatenate([dest, jnp.full((ep - e,), nn, jnp.int32)])

    zn = jnp.concatenate([z, n], axis=1)
    q0p = jnp.pad(q_0, ((0, 0), (0, 13)))
    zeros = jnp.zeros(((nn // 2 + 16) // 16, H), jnp.float32)

    blk_n = 2000
    blk_e = 4096

    x0 = _node_encode(zn, params["enc_node"], blk_n)
    gqs, gqd = _sc_gather_pair(q0p, src_p, dest_p)
    gxs, gxd = _sc_gather_pair(x0, src_p, dest_p)
    ea0, eattr1 = _edge_pass0(gqs, gqd, gxs, gxd,
                              params["enc_edge"], params["edge_0"], blk_e)
    agg0 = _sc_scatter_add(ea0, dest_s, zeros, nn)
    x1 = _node_update(x0, agg0, params["node_0"], None, blk_n)
    gxs1, gxd1 = _sc_gather_pair(x1, src_p, dest_p)
    ea1 = _edge_pass1(eattr1, gxs1, gxd1, params["edge_1"], blk_e)
    agg1 = _sc_scatter_add(ea1, dest_s, zeros, nn)
    return _node_update(x1, agg1, params["node_1"], params["dec"], blk_n)
